# baseline (speedup 1.0000x reference)
import functools

import jax
import jax.numpy as jnp
from jax import lax
from jax.experimental import pallas as pl
from jax.experimental.pallas import tpu as pltpu
from jax.experimental.pallas import tpu_sc as plsc

_PARAMS = pltpu.CompilerParams(
    use_tc_tiling_on_sc=True, needs_layout_passes=False)
_PARAMS_LINEAR = pltpu.CompilerParams(
    use_tc_tiling_on_sc=False, needs_layout_passes=False)


def _iota16():
    return lax.iota(jnp.int32, 16)


def _make_repack(V, D):
    info = plsc.get_sparse_core_info()
    NC, NS = info.num_cores, info.num_subcores
    NW = NC * NS
    n_full = V // 256
    tail = V - n_full * 256
    bpw = n_full // NW
    n_extra = n_full - bpw * NW
    assert bpw >= 4 and bpw % 2 == 0
    mesh = plsc.VectorSubcoreMesh(core_axis_name="c", subcore_axis_name="s")

    @functools.partial(
        pl.kernel,
        mesh=mesh,
        out_type=jax.ShapeDtypeStruct((V // 2, 2 * D), jnp.float32),
        scratch_types=[
            pltpu.VMEM((D, 256), jnp.float32),
            pltpu.VMEM((D, 256), jnp.float32),
            pltpu.VMEM((128, 2 * D), jnp.float32),
            pltpu.VMEM((128, 2 * D), jnp.float32),
            pltpu.VMEM((D, 64), jnp.float32),
            pltpu.VMEM((32, 2 * D), jnp.float32),
            pltpu.SemaphoreType.DMA,
            pltpu.SemaphoreType.DMA,
            pltpu.SemaphoreType.DMA,
            pltpu.SemaphoreType.DMA,
        ],
        compiler_params=_PARAMS,
    )
    def repack_kernel(wt_hbm, w128_hbm, in0, in1, ou0, ou1, int_, outt,
                      is0, is1, os0, os1):
        ibuf = (in0, in1)
        obuf = (ou0, ou1)
        isem = (is0, is1)
        osem = (os0, os1)
        wid = lax.axis_index("s") * NC + lax.axis_index("c")
        iota = _iota16()
        blk0 = wid * bpw

        def in_src(blk):
            return wt_hbm.at[:, pl.ds(blk * 256, 256)]

        def out_dst(blk):
            return w128_hbm.at[pl.ds(blk * 128, 128), :]

        def wait_in(b):
            pltpu.make_async_copy(in_src(blk0), ibuf[b], isem[b]).wait()

        def wait_out(b):
            pltpu.make_async_copy(obuf[b], out_dst(blk0), osem[b]).wait()

        def transpose(src, dst, ncol):
            dvecs = [iota + dg * 16 for dg in range(D // 16)]

            @plsc.parallel_loop(0, ncol, step=1, unroll=16)
            def _(c0):
                cc = lax.rem(c0 + iota, jnp.int32(ncol))
                qv = lax.shift_right_logical(cc, 1)
                zpar = lax.mul(lax.rem(cc, 2), jnp.int32(D))
                for dg in range(D // 16):
                    vals = plsc.load_gather(src, [dvecs[dg], cc])
                    plsc.store_scatter(dst, [qv, zpar + dvecs[dg]], vals)

        pltpu.async_copy(in_src(blk0), ibuf[0], isem[0])
        pltpu.async_copy(in_src(blk0 + 1), ibuf[1], isem[1])
        wait_in(0)
        transpose(ibuf[0], obuf[0], 256)
        pltpu.async_copy(obuf[0], out_dst(blk0), osem[0])
        pltpu.async_copy(in_src(blk0 + 2), ibuf[0], isem[0])
        wait_in(1)
        transpose(ibuf[1], obuf[1], 256)
        pltpu.async_copy(obuf[1], out_dst(blk0 + 1), osem[1])
        pltpu.async_copy(in_src(blk0 + 3), ibuf[1], isem[1])

        def body(j, carry):
            for b in (0, 1):
                m = 2 + 2 * j + b
                wait_in(b)
                wait_out(b)
                transpose(ibuf[b], obuf[b], 256)
                pltpu.async_copy(obuf[b], out_dst(blk0 + m), osem[b])
                m_next = jnp.minimum(m + 2, bpw - 1)
                pltpu.async_copy(in_src(blk0 + m_next), ibuf[b], isem[b])
            return carry

        lax.fori_loop(0, (bpw - 2) // 2, body, 0, unroll=False)

        wait_in(0)
        wait_in(1)
        wait_out(0)
        wait_out(1)

        if n_extra:
            @pl.when(wid < n_extra)
            def _():
                blk = NW * bpw + wid
                pltpu.sync_copy(in_src(blk), ibuf[0])
                transpose(ibuf[0], obuf[0], 256)
                pltpu.sync_copy(obuf[0], out_dst(blk))

        if tail:
            @pl.when(wid == n_extra)
            def _():
                pltpu.sync_copy(
                    wt_hbm.at[:, pl.ds(n_full * 256, tail)], int_)
                transpose(int_, outt, tail)
                pltpu.sync_copy(
                    outt, w128_hbm.at[pl.ds(n_full * 128, tail // 2), :])

    return repack_kernel


def _make_gather(V, D, S, P):
    info = plsc.get_sparse_core_info()
    NC, NS = info.num_cores, info.num_subcores
    NW = NC * NS
    n_ah = S // 256
    n_blocks = P * n_ah
    assert n_blocks % NW == 0
    n_per_w = n_blocks // NW
    assert n_per_w >= 4 and n_per_w % 2 == 0
    mesh = plsc.VectorSubcoreMesh(core_axis_name="c", subcore_axis_name="s")

    @functools.partial(
        pl.kernel,
        mesh=mesh,
        out_type=jax.ShapeDtypeStruct((P, D // 8, S // 128, 8, 128),
                                      jnp.float32),
        scratch_types=[
            pltpu.VMEM((256,), jnp.int32),
            pltpu.VMEM((256,), jnp.int32),
            pltpu.VMEM((256, D), jnp.float32),
            pltpu.VMEM((256, D), jnp.float32),
            pltpu.VMEM((D // 8, 2, 8, 128), jnp.float32),
            pltpu.VMEM((D // 8, 2, 8, 128), jnp.float32),
            pltpu.SemaphoreType.DMA,
            pltpu.SemaphoreType.DMA,
            pltpu.SemaphoreType.DMA,
            pltpu.SemaphoreType.DMA,
            pltpu.SemaphoreType.DMA,
            pltpu.SemaphoreType.DMA,
        ],
        compiler_params=_PARAMS_LINEAR,
    )
    def gather_kernel(w64_hbm, xt_hbm, out_hbm,
                      ib0, ib1, rb0, rb1, tb0, tb1,
                      is0, is1, gs0, gs1, os0, os1):
        ibuf = (ib0, ib1)
        rbuf = (rb0, rb1)
        tbuf = (tb0, tb1)
        isem = (is0, is1)
        gsem = (gs0, gs1)
        osem = (os0, os1)
        wid = lax.axis_index("s") * NC + lax.axis_index("c")
        iota = _iota16()
        k0 = wid * n_per_w
        albase = [iota + g * 16 for g in range(16)]

        def idx_src(k):
            b = k // n_ah
            ah = k - b * n_ah
            return xt_hbm.at[b, pl.ds(ah * 256, 256)]

        def out_dst(k):
            b = k // n_ah
            ah = k - b * n_ah
            return out_hbm.at[b, :, pl.ds(ah * 2, 2)]

        def wait_idx(b):
            pltpu.make_async_copy(idx_src(k0), ibuf[b], isem[b]).wait()

        def wait_gather(b):
            pltpu.make_async_copy(
                w64_hbm.at[ibuf[b]], rbuf[b], gsem[b]).wait()

        def wait_out(b):
            pltpu.make_async_copy(
                tbuf[b], out_hbm.at[0, :, pl.ds(0, 2)], osem[b]).wait()

        def transpose(b):
            rb = rbuf[b]
            tb = tbuf[b]

            @plsc.parallel_loop(0, D, step=1, unroll=16)
            def _(c0):
                cc = lax.rem(c0 + iota, jnp.int32(D))
                ch = lax.shift_right_logical(cc, 3)
                cl = lax.rem(cc, 8)
                for g in range(16):
                    ahl = iota * 0 + (g // 8)
                    al = iota + (g % 8) * 16
                    vals = plsc.load_gather(rb, [albase[g], cc])
                    plsc.store_scatter(tb, [ch, ahl, cl, al], vals)

        pltpu.async_copy(idx_src(k0), ibuf[0], isem[0])
        pltpu.async_copy(idx_src(k0 + 1), ibuf[1], isem[1])
        wait_idx(0)
        pltpu.async_copy(w64_hbm.at[ibuf[0]], rbuf[0], gsem[0])
        wait_idx(1)
        pltpu.async_copy(w64_hbm.at[ibuf[1]], rbuf[1], gsem[1])

        def stage(b, k, prefetch, wait_o):
            wait_gather(b)
            if wait_o:
                wait_out(b)
            transpose(b)
            pltpu.async_copy(tbuf[b], out_dst(k), osem[b])
            if prefetch:
                kn = jnp.minimum(k + 2, k0 + n_per_w - 1)
                pltpu.async_copy(idx_src(kn), ibuf[b], isem[b])
                wait_idx(b)
                pltpu.async_copy(w64_hbm.at[ibuf[b]], rbuf[b], gsem[b])

        stage(0, k0, True, False)
        stage(1, k0 + 1, True, False)

        def body(j, carry):
            for b in (0, 1):
                k = k0 + 2 + 2 * j + b
                stage(b, k, True, True)
            return carry

        lax.fori_loop(0, (n_per_w - 4) // 2, body, 0, unroll=False)

        stage(0, k0 + n_per_w - 2, False, True)
        stage(1, k0 + n_per_w - 1, False, True)
        wait_out(0)
        wait_out(1)

    return gather_kernel


def kernel(x, weight):
    V, D = weight.shape
    A, P = x.shape
    wt = weight.T
    xt = x.T.astype(jnp.int32)
    w128 = _make_repack(V, D)(wt)
    w64 = w128.reshape(V, D)
    out5 = _make_gather(V, D, A, P)(w64, xt)
    return out5.transpose(2, 4, 0, 1, 3).reshape(A, P, D)


# speedup vs baseline: 1.0037x; 1.0037x over previous
import functools

import jax
import jax.numpy as jnp
from jax import lax
from jax.experimental import pallas as pl
from jax.experimental.pallas import tpu as pltpu
from jax.experimental.pallas import tpu_sc as plsc

_PARAMS = pltpu.CompilerParams(
    use_tc_tiling_on_sc=True, needs_layout_passes=False)
_PARAMS_LINEAR = pltpu.CompilerParams(
    use_tc_tiling_on_sc=False, needs_layout_passes=False)


def _iota16():
    return lax.iota(jnp.int32, 16)


def _make_repack(V, D):
    info = plsc.get_sparse_core_info()
    NC, NS = info.num_cores, info.num_subcores
    NW = NC * NS
    n_full = V // 256
    tail = V - n_full * 256
    bpw = n_full // NW
    n_extra = n_full - bpw * NW
    assert bpw >= 4 and bpw % 2 == 0
    mesh = plsc.VectorSubcoreMesh(core_axis_name="c", subcore_axis_name="s")

    @functools.partial(
        pl.kernel,
        mesh=mesh,
        out_type=jax.ShapeDtypeStruct((V // 2, 2 * D), jnp.float32),
        scratch_types=[
            pltpu.VMEM((D, 256), jnp.float32),
            pltpu.VMEM((D, 256), jnp.float32),
            pltpu.VMEM((128, 2 * D), jnp.float32),
            pltpu.VMEM((128, 2 * D), jnp.float32),
            pltpu.VMEM((D, 64), jnp.float32),
            pltpu.VMEM((32, 2 * D), jnp.float32),
            pltpu.SemaphoreType.DMA,
            pltpu.SemaphoreType.DMA,
            pltpu.SemaphoreType.DMA,
            pltpu.SemaphoreType.DMA,
        ],
        compiler_params=_PARAMS,
    )
    def repack_kernel(wt_hbm, w128_hbm, in0, in1, ou0, ou1, int_, outt,
                      is0, is1, os0, os1):
        ibuf = (in0, in1)
        obuf = (ou0, ou1)
        isem = (is0, is1)
        osem = (os0, os1)
        wid = lax.axis_index("s") * NC + lax.axis_index("c")
        iota = _iota16()
        blk0 = wid * bpw

        def in_src(blk):
            return wt_hbm.at[:, pl.ds(blk * 256, 256)]

        def out_dst(blk):
            return w128_hbm.at[pl.ds(blk * 128, 128), :]

        def wait_in(b):
            pltpu.make_async_copy(in_src(blk0), ibuf[b], isem[b]).wait()

        def wait_out(b):
            pltpu.make_async_copy(obuf[b], out_dst(blk0), osem[b]).wait()

        def transpose(src, dst, ncol):
            dvecs = [iota + dg * 16 for dg in range(D // 16)]

            @plsc.parallel_loop(0, ncol, step=1, unroll=8)
            def _(c0):
                cc = lax.rem(c0 + iota, jnp.int32(ncol))
                qv = lax.shift_right_logical(cc, 1)
                zpar = lax.mul(lax.rem(cc, 2), jnp.int32(D))
                for dg in range(D // 16):
                    vals = plsc.load_gather(src, [dvecs[dg], cc])
                    plsc.store_scatter(dst, [qv, zpar + dvecs[dg]], vals)

        pltpu.async_copy(in_src(blk0), ibuf[0], isem[0])
        pltpu.async_copy(in_src(blk0 + 1), ibuf[1], isem[1])
        wait_in(0)
        transpose(ibuf[0], obuf[0], 256)
        pltpu.async_copy(obuf[0], out_dst(blk0), osem[0])
        pltpu.async_copy(in_src(blk0 + 2), ibuf[0], isem[0])
        wait_in(1)
        transpose(ibuf[1], obuf[1], 256)
        pltpu.async_copy(obuf[1], out_dst(blk0 + 1), osem[1])
        pltpu.async_copy(in_src(blk0 + 3), ibuf[1], isem[1])

        def body(j, carry):
            for b in (0, 1):
                m = 2 + 2 * j + b
                wait_in(b)
                wait_out(b)
                transpose(ibuf[b], obuf[b], 256)
                pltpu.async_copy(obuf[b], out_dst(blk0 + m), osem[b])
                m_next = jnp.minimum(m + 2, bpw - 1)
                pltpu.async_copy(in_src(blk0 + m_next), ibuf[b], isem[b])
            return carry

        lax.fori_loop(0, (bpw - 2) // 2, body, 0, unroll=False)

        wait_in(0)
        wait_in(1)
        wait_out(0)
        wait_out(1)

        if n_extra:
            @pl.when(wid < n_extra)
            def _():
                blk = NW * bpw + wid
                pltpu.sync_copy(in_src(blk), ibuf[0])
                transpose(ibuf[0], obuf[0], 256)
                pltpu.sync_copy(obuf[0], out_dst(blk))

        if tail:
            @pl.when(wid == n_extra)
            def _():
                pltpu.sync_copy(
                    wt_hbm.at[:, pl.ds(n_full * 256, tail)], int_)
                transpose(int_, outt, tail)
                pltpu.sync_copy(
                    outt, w128_hbm.at[pl.ds(n_full * 128, tail // 2), :])

    return repack_kernel


def _make_gather(V, D, S, P):
    info = plsc.get_sparse_core_info()
    NC, NS = info.num_cores, info.num_subcores
    NW = NC * NS
    n_ah = S // 256
    n_blocks = P * n_ah
    assert n_blocks % NW == 0
    n_per_w = n_blocks // NW
    assert n_per_w >= 4 and n_per_w % 2 == 0
    mesh = plsc.VectorSubcoreMesh(core_axis_name="c", subcore_axis_name="s")

    @functools.partial(
        pl.kernel,
        mesh=mesh,
        out_type=jax.ShapeDtypeStruct((P, D // 8, S // 128, 8, 128),
                                      jnp.float32),
        scratch_types=[
            pltpu.VMEM((256,), jnp.int32),
            pltpu.VMEM((256,), jnp.int32),
            pltpu.VMEM((256, D), jnp.float32),
            pltpu.VMEM((256, D), jnp.float32),
            pltpu.VMEM((D // 8, 2, 8, 128), jnp.float32),
            pltpu.VMEM((D // 8, 2, 8, 128), jnp.float32),
            pltpu.SemaphoreType.DMA,
            pltpu.SemaphoreType.DMA,
            pltpu.SemaphoreType.DMA,
            pltpu.SemaphoreType.DMA,
            pltpu.SemaphoreType.DMA,
            pltpu.SemaphoreType.DMA,
        ],
        compiler_params=_PARAMS_LINEAR,
    )
    def gather_kernel(w64_hbm, xt_hbm, out_hbm,
                      ib0, ib1, rb0, rb1, tb0, tb1,
                      is0, is1, gs0, gs1, os0, os1):
        ibuf = (ib0, ib1)
        rbuf = (rb0, rb1)
        tbuf = (tb0, tb1)
        isem = (is0, is1)
        gsem = (gs0, gs1)
        osem = (os0, os1)
        wid = lax.axis_index("s") * NC + lax.axis_index("c")
        iota = _iota16()
        k0 = wid * n_per_w
        albase = [iota + g * 16 for g in range(16)]

        def idx_src(k):
            b = k // n_ah
            ah = k - b * n_ah
            return xt_hbm.at[b, pl.ds(ah * 256, 256)]

        def out_dst(k):
            b = k // n_ah
            ah = k - b * n_ah
            return out_hbm.at[b, :, pl.ds(ah * 2, 2)]

        def wait_idx(b):
            pltpu.make_async_copy(idx_src(k0), ibuf[b], isem[b]).wait()

        def wait_gather(b):
            pltpu.make_async_copy(
                w64_hbm.at[ibuf[b]], rbuf[b], gsem[b]).wait()

        def wait_out(b):
            pltpu.make_async_copy(
                tbuf[b], out_hbm.at[0, :, pl.ds(0, 2)], osem[b]).wait()

        def transpose(b):
            rb = rbuf[b]
            tb = tbuf[b]

            @plsc.parallel_loop(0, D, step=1, unroll=8)
            def _(c0):
                cc = lax.rem(c0 + iota, jnp.int32(D))
                ch = lax.shift_right_logical(cc, 3)
                cl = lax.rem(cc, 8)
                for g in range(16):
                    ahl = iota * 0 + (g // 8)
                    al = iota + (g % 8) * 16
                    vals = plsc.load_gather(rb, [albase[g], cc])
                    plsc.store_scatter(tb, [ch, ahl, cl, al], vals)

        pltpu.async_copy(idx_src(k0), ibuf[0], isem[0])
        pltpu.async_copy(idx_src(k0 + 1), ibuf[1], isem[1])
        wait_idx(0)
        pltpu.async_copy(w64_hbm.at[ibuf[0]], rbuf[0], gsem[0])
        wait_idx(1)
        pltpu.async_copy(w64_hbm.at[ibuf[1]], rbuf[1], gsem[1])

        def stage(b, k, prefetch, wait_o):
            wait_gather(b)
            if wait_o:
                wait_out(b)
            transpose(b)
            pltpu.async_copy(tbuf[b], out_dst(k), osem[b])
            if prefetch:
                kn = jnp.minimum(k + 2, k0 + n_per_w - 1)
                pltpu.async_copy(idx_src(kn), ibuf[b], isem[b])
                wait_idx(b)
                pltpu.async_copy(w64_hbm.at[ibuf[b]], rbuf[b], gsem[b])

        stage(0, k0, True, False)
        stage(1, k0 + 1, True, False)

        def body(j, carry):
            for b in (0, 1):
                k = k0 + 2 + 2 * j + b
                stage(b, k, True, True)
            return carry

        lax.fori_loop(0, (n_per_w - 4) // 2, body, 0, unroll=False)

        stage(0, k0 + n_per_w - 2, False, True)
        stage(1, k0 + n_per_w - 1, False, True)
        wait_out(0)
        wait_out(1)

    return gather_kernel


def kernel(x, weight):
    V, D = weight.shape
    A, P = x.shape
    wt = weight.T
    xt = x.T.astype(jnp.int32)
    w128 = _make_repack(V, D)(wt)
    w64 = w128.reshape(V, D)
    out5 = _make_gather(V, D, A, P)(w64, xt)
    return out5.transpose(2, 4, 0, 1, 3).reshape(A, P, D)
